# baseline (speedup 1.0000x reference)
import jax
import jax.numpy as jnp
from jax.experimental import pallas as pl
from jax.experimental.pallas import tpu as pltpu

H = 768
F = 192
E = 64
NS = 2
FS = F * 2
K = 8
T = 2048
G = 4
NG = E // G


def _moe_kernel(xf_ref, wr_ref, wg_ref, wu_ref, wd_ref,
                wgs_ref, wus_ref, wds_ref, out_ref, aux_ref, cnt_ref,
                w_ref, hw_ref, xb_ref):
    i = pl.program_id(0)

    @pl.when(i == 0)
    def _router():
        out_ref[...] = jnp.zeros_like(out_ref)
        cnt_ref[...] = jnp.zeros_like(cnt_ref)
        x = xf_ref[...]
        xb_ref[...] = x.astype(jnp.bfloat16)
        logits = jnp.dot(x, wr_ref[...], preferred_element_type=jnp.float32)
        m = jnp.max(logits, axis=-1, keepdims=True)
        ex = jnp.exp(logits - m)
        probs = ex / jnp.sum(ex, axis=-1, keepdims=True)
        p = probs
        acc = jnp.zeros_like(probs)
        ssum = jnp.zeros((T, 1), jnp.float32)
        iota = jax.lax.broadcasted_iota(jnp.int32, (T, E), 1)
        for _ in range(K):
            mk = jnp.max(p, axis=-1, keepdims=True)
            idx = jnp.min(jnp.where(p == mk, iota, E), axis=-1, keepdims=True)
            sel = iota == idx
            acc = acc + jnp.where(sel, mk, 0.0)
            ssum = ssum + mk
            p = jnp.where(sel, -1.0, p)
        weights = acc / ssum
        w_ref[...] = weights
        maskf = (weights > 0).astype(jnp.float32)
        aux_ref[...] = E * jnp.sum(
            jnp.mean(probs, axis=0, keepdims=True)
            * jnp.mean(maskf, axis=0, keepdims=True),
            axis=1, keepdims=True)

    @pl.when((i > 0) & (i <= NG))
    def _down():
        wd4 = wd_ref[...].reshape(G * F, H).astype(jnp.bfloat16)
        out_ref[...] += jnp.dot(hw_ref[...], wd4,
                                preferred_element_type=jnp.float32)

    @pl.when(i < NG)
    def _experts():
        x = xb_ref[...]
        e0 = G * i
        rhs = jnp.concatenate(
            [wg_ref[j].astype(jnp.bfloat16) for j in range(G)]
            + [wu_ref[j].astype(jnp.bfloat16) for j in range(G)],
            axis=1)
        gu = jnp.dot(x, rhs, preferred_element_type=jnp.float32)
        g4 = gu[:, :G * F]
        u4 = gu[:, G * F:]
        hid4 = jnp.where(g4 > 0.0, g4, 0.0) * u4
        cnt_ref[...] += jnp.sum((hid4 == 0.0).astype(jnp.float32),
                                keepdims=True).reshape(1, 1)
        ei = jax.lax.broadcasted_iota(jnp.int32, (E, F), 0)
        sel4 = jnp.concatenate(
            [(ei == e0 + j).astype(jnp.float32) for j in range(G)],
            axis=1)
        wb4 = jnp.dot(w_ref[...], sel4, preferred_element_type=jnp.float32)
        hw_ref[...] = (hid4 * wb4).astype(jnp.bfloat16)

    @pl.when(i == NG)
    def _shared():
        x = xb_ref[...]
        rhs = jnp.concatenate(
            [wgs_ref[j].astype(jnp.bfloat16) for j in range(NS)]
            + [wus_ref[j].astype(jnp.bfloat16) for j in range(NS)],
            axis=1)
        gus = jnp.dot(x, rhs, preferred_element_type=jnp.float32)
        gs2 = gus[:, :NS * FS]
        us2 = gus[:, NS * FS:]
        hs2 = (jnp.where(gs2 > 0.0, gs2, 0.0) * us2).astype(jnp.bfloat16)
        wds2 = wds_ref[...].reshape(NS * FS, H).astype(jnp.bfloat16)
        out_ref[...] += jnp.dot(hs2, wds2, preferred_element_type=jnp.float32)


def kernel(x, Wr, Wg, Wu, Wd, Wg_s, Wu_s, Wd_s):
    b, s, h = x.shape
    xf = x.reshape(-1, h)

    gidx = lambda i: (jnp.minimum(i, NG - 1), 0, 0)
    didx = lambda i: (jnp.clip(i - 1, 0, NG - 1), 0, 0)
    out, aux, cnt = pl.pallas_call(
        _moe_kernel,
        grid=(NG + 1,),
        in_specs=[
            pl.BlockSpec((T, H), lambda i: (0, 0)),
            pl.BlockSpec((H, E), lambda i: (0, 0)),
            pl.BlockSpec((G, H, F), gidx),
            pl.BlockSpec((G, H, F), gidx),
            pl.BlockSpec((G, F, H), didx),
            pl.BlockSpec((NS, H, FS), lambda i: (0, 0, 0)),
            pl.BlockSpec((NS, H, FS), lambda i: (0, 0, 0)),
            pl.BlockSpec((NS, FS, H), lambda i: (0, 0, 0)),
        ],
        out_specs=(
            pl.BlockSpec((T, H), lambda i: (0, 0)),
            pl.BlockSpec((1, 1), lambda i: (0, 0)),
            pl.BlockSpec((1, 1), lambda i: (0, 0)),
        ),
        out_shape=(
            jax.ShapeDtypeStruct((T, H), jnp.float32),
            jax.ShapeDtypeStruct((1, 1), jnp.float32),
            jax.ShapeDtypeStruct((1, 1), jnp.float32),
        ),
        scratch_shapes=[
            pltpu.VMEM((T, E), jnp.float32),
            pltpu.VMEM((T, G * F), jnp.bfloat16),
            pltpu.VMEM((T, H), jnp.bfloat16),
        ],
    )(xf, Wr, Wg, Wu, Wd, Wg_s, Wu_s, Wd_s)

    sparsity = (cnt[0, 0] / (E * T * F)).reshape(())
    return (out.reshape(b, s, h), aux.reshape(()), sparsity)


# speedup vs baseline: 1.0091x; 1.0091x over previous
import jax
import jax.numpy as jnp
from jax.experimental import pallas as pl
from jax.experimental.pallas import tpu as pltpu

H = 768
F = 192
E = 64
NS = 2
FS = F * 2
K = 8
T = 2048
G = 4
NG = E // G


def _moe_kernel(xf_ref, wr_ref, wg_ref, wu_ref, wd_ref,
                wgs_ref, wus_ref, wds_ref, out_ref, aux_ref, cnt_ref,
                w_ref, hw_ref, xb_ref):
    i = pl.program_id(0)

    @pl.when(i == 0)
    def _router():
        out_ref[...] = jnp.zeros_like(out_ref)
        cnt_ref[...] = jnp.zeros_like(cnt_ref)
        x = xf_ref[...]
        xb_ref[...] = x.astype(jnp.bfloat16)
        logits = jnp.dot(x, wr_ref[...], preferred_element_type=jnp.float32)
        m = jnp.max(logits, axis=-1, keepdims=True)
        ex = jnp.exp(logits - m)
        probs = ex / jnp.sum(ex, axis=-1, keepdims=True)
        p = probs
        acc = jnp.zeros_like(probs)
        ssum = jnp.zeros((T, 1), jnp.float32)
        iota = jax.lax.broadcasted_iota(jnp.int32, (T, E), 1)
        for _ in range(K):
            mk = jnp.max(p, axis=-1, keepdims=True)
            idx = jnp.min(jnp.where(p == mk, iota, E), axis=-1, keepdims=True)
            sel = iota == idx
            acc = acc + jnp.where(sel, mk, 0.0)
            ssum = ssum + mk
            p = jnp.where(sel, -1.0, p)
        weights = acc / ssum
        w_ref[...] = weights
        maskf = (weights > 0).astype(jnp.float32)
        aux_ref[...] = E * jnp.sum(
            jnp.mean(probs, axis=0, keepdims=True)
            * jnp.mean(maskf, axis=0, keepdims=True),
            axis=1, keepdims=True)

    @pl.when((i > 1) & (i <= NG + 1))
    def _down():
        wd4 = wd_ref[...].reshape(G * F, H).astype(jnp.bfloat16)
        out_ref[...] += jnp.dot(hw_ref[...], wd4,
                                preferred_element_type=jnp.float32)

    @pl.when((i > 0) & (i <= NG))
    def _experts():
        x = xb_ref[...]
        e0 = G * (i - 1)
        rhs = jnp.concatenate(
            [wg_ref[j].astype(jnp.bfloat16) for j in range(G)]
            + [wu_ref[j].astype(jnp.bfloat16) for j in range(G)],
            axis=1)
        gu = jnp.dot(x, rhs, preferred_element_type=jnp.float32)
        g4 = gu[:, :G * F]
        u4 = gu[:, G * F:]
        hid4 = jnp.where(g4 > 0.0, g4, 0.0) * u4
        cnt_ref[...] += jnp.sum((hid4 == 0.0).astype(jnp.float32),
                                keepdims=True).reshape(1, 1)
        ei = jax.lax.broadcasted_iota(jnp.int32, (E, F), 0)
        sel4 = jnp.concatenate(
            [(ei == e0 + j).astype(jnp.float32) for j in range(G)],
            axis=1)
        wb4 = jnp.dot(w_ref[...], sel4, preferred_element_type=jnp.float32)
        hw_ref[...] = (hid4 * wb4).astype(jnp.bfloat16)

    @pl.when(i == NG + 1)
    def _shared():
        x = xb_ref[...]
        rhs = jnp.concatenate(
            [wgs_ref[j].astype(jnp.bfloat16) for j in range(NS)]
            + [wus_ref[j].astype(jnp.bfloat16) for j in range(NS)],
            axis=1)
        gus = jnp.dot(x, rhs, preferred_element_type=jnp.float32)
        gs2 = gus[:, :NS * FS]
        us2 = gus[:, NS * FS:]
        hs2 = (jnp.where(gs2 > 0.0, gs2, 0.0) * us2).astype(jnp.bfloat16)
        wds2 = wds_ref[...].reshape(NS * FS, H).astype(jnp.bfloat16)
        out_ref[...] += jnp.dot(hs2, wds2, preferred_element_type=jnp.float32)


def kernel(x, Wr, Wg, Wu, Wd, Wg_s, Wu_s, Wd_s):
    b, s, h = x.shape
    xf = x.reshape(-1, h)

    gidx = lambda i: (jnp.clip(i - 1, 0, NG - 1), 0, 0)
    didx = lambda i: (jnp.clip(i - 2, 0, NG - 1), 0, 0)
    out, aux, cnt = pl.pallas_call(
        _moe_kernel,
        grid=(NG + 2,),
        in_specs=[
            pl.BlockSpec((T, H), lambda i: (0, 0)),
            pl.BlockSpec((H, E), lambda i: (0, 0)),
            pl.BlockSpec((G, H, F), gidx),
            pl.BlockSpec((G, H, F), gidx),
            pl.BlockSpec((G, F, H), didx),
            pl.BlockSpec((NS, H, FS), lambda i: (0, 0, 0)),
            pl.BlockSpec((NS, H, FS), lambda i: (0, 0, 0)),
            pl.BlockSpec((NS, FS, H), lambda i: (0, 0, 0)),
        ],
        out_specs=(
            pl.BlockSpec((T, H), lambda i: (0, 0)),
            pl.BlockSpec((1, 1), lambda i: (0, 0)),
            pl.BlockSpec((1, 1), lambda i: (0, 0)),
        ),
        out_shape=(
            jax.ShapeDtypeStruct((T, H), jnp.float32),
            jax.ShapeDtypeStruct((1, 1), jnp.float32),
            jax.ShapeDtypeStruct((1, 1), jnp.float32),
        ),
        scratch_shapes=[
            pltpu.VMEM((T, E), jnp.float32),
            pltpu.VMEM((T, G * F), jnp.bfloat16),
            pltpu.VMEM((T, H), jnp.bfloat16),
        ],
    )(xf, Wr, Wg, Wu, Wd, Wg_s, Wu_s, Wd_s)

    sparsity = (cnt[0, 0] / (E * T * F)).reshape(())
    return (out.reshape(b, s, h), aux.reshape(()), sparsity)
